# Initial kernel scaffold; baseline (speedup 1.0000x reference)
import functools

import jax
import jax.numpy as jnp
from jax import lax
from jax.experimental import pallas as pl
from jax.experimental.pallas import tpu as pltpu
from jax.experimental.pallas import tpu_sc as plsc

N_NODES = 10000
N_EDGES = 320000
D = 128

NC = 2
NS = 16
NW = NC * NS
E_PER_W = N_EDGES // NW
CHUNK = 125
N_CHUNKS = E_PER_W // CHUNK
ROWS_PER_TILE = N_NODES // NS
DEG_W = 16


def _sc_aggregate(feature, src3, dst3, zero_acc, zero_deg, ones_rows):
    mesh = plsc.VectorSubcoreMesh(core_axis_name="c", subcore_axis_name="s")

    @functools.partial(
        pl.kernel,
        out_type=(
            jax.ShapeDtypeStruct((NC, N_NODES, D), jnp.float32),
            jax.ShapeDtypeStruct((NC, N_NODES, DEG_W), jnp.float32),
        ),
        mesh=mesh,
        scratch_types=[
            pltpu.VMEM((N_CHUNKS, CHUNK), jnp.int32),
            pltpu.VMEM((N_CHUNKS, CHUNK), jnp.int32),
            pltpu.VMEM((CHUNK, D), jnp.float32),
            pltpu.VMEM((CHUNK, DEG_W), jnp.float32),
            pltpu.VMEM_SHARED((N_NODES, D), jnp.float32),
            pltpu.VMEM_SHARED((N_NODES, DEG_W), jnp.float32),
            pltpu.SemaphoreType.DMA,
        ],
    )
    def body(feature_hbm, src_hbm, dst_hbm, zacc_hbm, zdeg_hbm, ones_hbm,
             acc_out, deg_out, src_v, dst_v, rows_v, ones_v, acc_sh, deg_sh,
             gsem):
        cid = lax.axis_index("c")
        sid = lax.axis_index("s")
        wid = cid * NS + sid

        pltpu.sync_copy(src_hbm.at[wid], src_v)
        pltpu.sync_copy(dst_hbm.at[wid], dst_v)
        pltpu.sync_copy(ones_hbm, ones_v)
        stripe = pl.ds(sid * ROWS_PER_TILE, ROWS_PER_TILE)
        pltpu.sync_copy(zacc_hbm, acc_sh.at[stripe])
        pltpu.sync_copy(zdeg_hbm, deg_sh.at[stripe])
        plsc.subcore_barrier()

        def step(j, carry):
            pltpu.async_copy(feature_hbm.at[src_v.at[j]], rows_v, gsem).wait()
            pltpu.sync_copy(rows_v, acc_sh.at[dst_v.at[j]], add=True)
            pltpu.sync_copy(ones_v, deg_sh.at[dst_v.at[j]], add=True)
            return carry

        lax.fori_loop(0, N_CHUNKS, step, 0)
        plsc.subcore_barrier()

        pltpu.sync_copy(acc_sh.at[stripe], acc_out.at[cid, stripe])
        pltpu.sync_copy(deg_sh.at[stripe], deg_out.at[cid, stripe])

    return body(feature, src3, dst3, zero_acc, zero_deg, ones_rows)


def _tc_finish_body(acc_ref, deg_ref, wt_ref, b_ref, out_ref):
    s = acc_ref[0] + acc_ref[1]
    d = deg_ref[0, :, 0:1] + deg_ref[1, :, 0:1]
    h = s / jnp.maximum(d, 1.0)
    o = jnp.dot(h, wt_ref[...], preferred_element_type=jnp.float32)
    out_ref[...] = jnp.maximum(o + b_ref[...], 0.0)


def _tc_finish(acc, deg, wt, b2d):
    R = 400
    grid = (N_NODES // R,)
    return pl.pallas_call(
        _tc_finish_body,
        grid=grid,
        in_specs=[
            pl.BlockSpec((NC, R, D), lambda i: (0, i, 0)),
            pl.BlockSpec((NC, R, DEG_W), lambda i: (0, i, 0)),
            pl.BlockSpec((D, D), lambda i: (0, 0)),
            pl.BlockSpec((1, D), lambda i: (0, 0)),
        ],
        out_specs=pl.BlockSpec((R, D), lambda i: (i, 0)),
        out_shape=jax.ShapeDtypeStruct((N_NODES, D), jnp.float32),
    )(acc, deg, wt, b2d)


def kernel(feature, edge_index, W, b):
    src = edge_index[0].astype(jnp.int32).reshape(NW, N_CHUNKS, CHUNK)
    dst = edge_index[1].astype(jnp.int32).reshape(NW, N_CHUNKS, CHUNK)
    zero_acc = jnp.zeros((ROWS_PER_TILE, D), jnp.float32)
    zero_deg = jnp.zeros((ROWS_PER_TILE, DEG_W), jnp.float32)
    ones_rows = jnp.ones((CHUNK, DEG_W), jnp.float32)

    acc, deg = _sc_aggregate(feature, src, dst, zero_acc, zero_deg, ones_rows)
    return _tc_finish(acc, deg, W.T, b.reshape(1, D))


# speedup vs baseline: 4.4498x; 4.4498x over previous
import functools

import jax
import jax.numpy as jnp
from jax import lax
from jax.experimental import pallas as pl
from jax.experimental.pallas import tpu as pltpu
from jax.experimental.pallas import tpu_sc as plsc

N_NODES = 10000
N_EDGES = 320000
D = 128
DA = 144

NC = 2
NS = 16
HALF = N_NODES // NC
NPC = 5120
E_PER_T = N_EDGES // NS
SEG = 10000
N_SEG = E_PER_T // SEG
CHUNK = 80
N_CHUNKS = SEG // CHUNK
STRIPE = NPC // NS
ZR = 80
DEG_W = 16
V = 16


def _sc_aggregate(feat_aug, src_e, dst_e, zero_acc):
    mesh = plsc.VectorSubcoreMesh(core_axis_name="c", subcore_axis_name="s")

    @functools.partial(
        pl.kernel,
        out_type=jax.ShapeDtypeStruct((NC, NPC, DA), jnp.float32),
        mesh=mesh,
        compiler_params=pltpu.CompilerParams(use_tc_tiling_on_sc=False),
        scratch_types=[
            pltpu.VMEM((SEG,), jnp.int32),
            pltpu.VMEM((SEG,), jnp.int32),
            pltpu.VMEM((CHUNK,), jnp.int32),
            pltpu.VMEM((CHUNK, DA), jnp.float32),
            pltpu.VMEM_SHARED((NPC, DA), jnp.float32),
            pltpu.SemaphoreType.DMA,
        ],
    )
    def body(feat_hbm, src_hbm, dst_hbm, zacc_hbm, acc_out,
             src_v, dst_v, chunk_v, rows_v, acc_sh, gsem):
        cid = lax.axis_index("c")
        sid = lax.axis_index("s")
        lo = cid * HALF

        for q in range(STRIPE // ZR):
            sub = pl.ds(sid * STRIPE + q * ZR, ZR)
            pltpu.sync_copy(zacc_hbm, acc_sh.at[sub])
        plsc.subcore_barrier()

        lane = lax.iota(jnp.int32, V)

        def chunk_step(j, carry):
            for v in range(CHUNK // V):
                local = dst_v[pl.ds(j * CHUNK + v * V, V)] - lo
                ok = (local >= 0) & (local < HALF)
                dump = HALF + ((j + v) & 63) + lane
                chunk_v[pl.ds(v * V, V)] = jnp.where(ok, local, dump)
            pltpu.async_copy(
                feat_hbm.at[src_v.at[pl.ds(j * CHUNK, CHUNK)]],
                rows_v, gsem).wait()
            pltpu.sync_copy(rows_v, acc_sh.at[chunk_v], add=True)
            return carry

        for s in range(N_SEG):
            base = sid * E_PER_T + s * SEG
            pltpu.sync_copy(src_hbm.at[pl.ds(base, SEG)], src_v)
            pltpu.sync_copy(dst_hbm.at[pl.ds(base, SEG)], dst_v)
            lax.fori_loop(0, N_CHUNKS, chunk_step, 0)

        plsc.subcore_barrier()
        for q in range(STRIPE // ZR):
            sub = pl.ds(sid * STRIPE + q * ZR, ZR)
            pltpu.sync_copy(acc_sh.at[sub], acc_out.at[cid, sub])

    return body(feat_aug, src_e, dst_e, zero_acc)


def _tc_finish_body(acc_ref, deg_ref, wt_ref, b_ref, out_ref):
    d = deg_ref[:, 0:1]
    h = acc_ref[...] / jnp.maximum(d, 1.0)
    o = jnp.dot(h, wt_ref[...], preferred_element_type=jnp.float32)
    out_ref[...] = jnp.maximum(o + b_ref[...], 0.0)


def _tc_finish(acc, deg, wt, b2d):
    R = 400
    grid = (N_NODES // R,)
    return pl.pallas_call(
        _tc_finish_body,
        grid=grid,
        in_specs=[
            pl.BlockSpec((R, D), lambda i: (i, 0)),
            pl.BlockSpec((R, DEG_W), lambda i: (i, 0)),
            pl.BlockSpec((D, D), lambda i: (0, 0)),
            pl.BlockSpec((1, D), lambda i: (0, 0)),
        ],
        out_specs=pl.BlockSpec((R, D), lambda i: (i, 0)),
        out_shape=jax.ShapeDtypeStruct((N_NODES, D), jnp.float32),
    )(acc, deg, wt, b2d)


def kernel(feature, edge_index, W, b):
    src_e = edge_index[0].astype(jnp.int32)
    dst_e = edge_index[1].astype(jnp.int32)
    feat_aug = jnp.concatenate(
        [feature,
         jnp.ones((N_NODES, 1), jnp.float32),
         jnp.zeros((N_NODES, DA - D - 1), jnp.float32)], axis=1)
    zero_acc = jnp.zeros((ZR, DA), jnp.float32)

    acc = _sc_aggregate(feat_aug, src_e, dst_e, zero_acc)
    acc_n = acc[:, :HALF, :D].reshape(N_NODES, D)
    deg_n = acc[:, :HALF, D:D + DEG_W].reshape(N_NODES, DEG_W)
    return _tc_finish(acc_n, deg_n, W.T, b.reshape(1, D))


# speedup vs baseline: 7.0925x; 1.5939x over previous
import functools

import jax
import jax.numpy as jnp
from jax import lax
from jax.experimental import pallas as pl
from jax.experimental.pallas import tpu as pltpu
from jax.experimental.pallas import tpu_sc as plsc

N_NODES = 10000
N_EDGES = 320000
D = 128
DA = 144

NC = 2
NS = 16
HALF = N_NODES // NC
NPC = 5120
E_PER_T = N_EDGES // NS
SEG = 10000
N_SEG = E_PER_T // SEG
CHUNK = 80
N_CHUNKS = SEG // CHUNK
STRIPE = NPC // NS
ZR = 80
DEG_W = 16
V = 16


def _sc_aggregate(feat_aug, src_e, dst_e, zero_acc):
    mesh = plsc.VectorSubcoreMesh(core_axis_name="c", subcore_axis_name="s")

    @functools.partial(
        pl.kernel,
        out_type=jax.ShapeDtypeStruct((NC, NPC, DA), jnp.float32),
        mesh=mesh,
        compiler_params=pltpu.CompilerParams(use_tc_tiling_on_sc=False),
        scratch_types=[
            pltpu.VMEM((SEG,), jnp.int32),
            pltpu.VMEM((SEG,), jnp.int32),
            pltpu.VMEM((CHUNK,), jnp.int32),
            pltpu.VMEM((CHUNK, DA), jnp.float32),
            pltpu.VMEM((CHUNK, DA), jnp.float32),
            pltpu.VMEM_SHARED((NPC, DA), jnp.float32),
            pltpu.SemaphoreType.DMA,
            pltpu.SemaphoreType.DMA,
        ],
    )
    def body(feat_hbm, src_hbm, dst_hbm, zacc_hbm, acc_out,
             src_v, dst_v, chunk_v, rows_a, rows_b, acc_sh, sem_a, sem_b):
        cid = lax.axis_index("c")
        sid = lax.axis_index("s")
        lo = cid * HALF

        for q in range(STRIPE // ZR):
            sub = pl.ds(sid * STRIPE + q * ZR, ZR)
            pltpu.sync_copy(zacc_hbm, acc_sh.at[sub])
        plsc.subcore_barrier()

        lane = lax.iota(jnp.int32, V)

        def remap(j):
            for v in range(CHUNK // V):
                local = dst_v[pl.ds(j * CHUNK + v * V, V)] - lo
                ok = (local >= 0) & (local < HALF)
                dump = HALF + ((j + v) & 63) + lane
                chunk_v[pl.ds(v * V, V)] = jnp.where(ok, local, dump)

        def gsrc(j):
            return feat_hbm.at[src_v.at[pl.ds(j * CHUNK, CHUNK)]]

        def pair_step(p, carry):
            j0 = 2 * p
            j1 = j0 + 1
            pltpu.async_copy(gsrc(j1), rows_b, sem_b)
            pltpu.make_async_copy(gsrc(j0), rows_a, sem_a).wait()
            remap(j0)
            pltpu.sync_copy(rows_a, acc_sh.at[chunk_v], add=True)
            pltpu.async_copy(gsrc(j0 + 2), rows_a, sem_a)
            pltpu.make_async_copy(gsrc(j1), rows_b, sem_b).wait()
            remap(j1)
            pltpu.sync_copy(rows_b, acc_sh.at[chunk_v], add=True)
            return carry

        last = N_CHUNKS - 1
        for s in range(N_SEG):
            base = sid * E_PER_T + s * SEG
            pltpu.sync_copy(src_hbm.at[pl.ds(base, SEG)], src_v)
            pltpu.sync_copy(dst_hbm.at[pl.ds(base, SEG)], dst_v)
            pltpu.async_copy(gsrc(0), rows_a, sem_a)
            lax.fori_loop(0, N_CHUNKS // 2, pair_step, 0)
            pltpu.make_async_copy(gsrc(last), rows_a, sem_a).wait()
            remap(last)
            pltpu.sync_copy(rows_a, acc_sh.at[chunk_v], add=True)

        plsc.subcore_barrier()
        for q in range(STRIPE // ZR):
            sub = pl.ds(sid * STRIPE + q * ZR, ZR)
            pltpu.sync_copy(acc_sh.at[sub], acc_out.at[cid, sub])

    return body(feat_aug, src_e, dst_e, zero_acc)


def _tc_finish_body(acc_ref, deg_ref, wt_ref, b_ref, out_ref):
    d = deg_ref[:, 0:1]
    h = acc_ref[...] / jnp.maximum(d, 1.0)
    o = jnp.dot(h, wt_ref[...], preferred_element_type=jnp.float32)
    out_ref[...] = jnp.maximum(o + b_ref[...], 0.0)


def _tc_finish(acc, deg, wt, b2d):
    R = 400
    grid = (N_NODES // R,)
    return pl.pallas_call(
        _tc_finish_body,
        grid=grid,
        in_specs=[
            pl.BlockSpec((R, D), lambda i: (i, 0)),
            pl.BlockSpec((R, DEG_W), lambda i: (i, 0)),
            pl.BlockSpec((D, D), lambda i: (0, 0)),
            pl.BlockSpec((1, D), lambda i: (0, 0)),
        ],
        out_specs=pl.BlockSpec((R, D), lambda i: (i, 0)),
        out_shape=jax.ShapeDtypeStruct((N_NODES, D), jnp.float32),
    )(acc, deg, wt, b2d)


def kernel(feature, edge_index, W, b):
    src_e = edge_index[0].astype(jnp.int32)
    dst_e = edge_index[1].astype(jnp.int32)
    feat_aug = jnp.concatenate(
        [feature,
         jnp.ones((N_NODES, 1), jnp.float32),
         jnp.zeros((N_NODES, DA - D - 1), jnp.float32)], axis=1)
    zero_acc = jnp.zeros((ZR, DA), jnp.float32)

    acc = _sc_aggregate(feat_aug, src_e, dst_e, zero_acc)
    acc_n = acc[:, :HALF, :D].reshape(N_NODES, D)
    deg_n = acc[:, :HALF, D:D + DEG_W].reshape(N_NODES, DEG_W)
    return _tc_finish(acc_n, deg_n, W.T, b.reshape(1, D))
